# baseline (speedup 1.0000x reference)
import functools

import jax
import jax.numpy as jnp
from jax import lax
from jax.experimental import pallas as pl
from jax.experimental.pallas import tpu as pltpu
from jax.experimental.pallas import tpu_sc as plsc

N_NODES = 10000
N_EDGES = 160000
D = 256

NC = 2
NS = 16
NW = NC * NS
EPW = N_EDGES // NW
CH = 160
NCHUNK = 32
TAIL_OFF = EPW - CH


def _chunk_off(c):
    return jnp.minimum(c * CH, TAIL_OFF)


def _dot_body(src_hbm, dst_hbm, h_hbm, out_hbm,
              idx_src_v, idx_dst_v,
              rows_s0, rows_d0, rows_s1, rows_d1,
              out0, out1, m_v,
              gsem0, gsem1, osem0, osem1):
    wid = lax.axis_index("s") * NC + lax.axis_index("c")
    base = wid * EPW
    pltpu.sync_copy(src_hbm.at[pl.ds(base, EPW)], idx_src_v)
    pltpu.sync_copy(dst_hbm.at[pl.ds(base, EPW)], idx_dst_v)

    lane = lax.iota(jnp.int32, 16)
    cols = [jnp.full((16,), k, jnp.int32) for k in range(16)]

    def issue(c, rs, rd, sem):
        off = _chunk_off(c)
        pltpu.async_copy(h_hbm.at[idx_src_v.at[pl.ds(off, CH)]], rs, sem)
        pltpu.async_copy(h_hbm.at[idx_dst_v.at[pl.ds(off, CH)]], rd, sem)

    def wait_rows(rs, rd, sem):
        dummy = h_hbm.at[pl.ds(0, CH), :]
        pltpu.make_async_copy(dummy, rs, sem).wait()
        pltpu.make_async_copy(dummy, rd, sem).wait()

    def wait_out(ob, sem):
        pltpu.make_async_copy(ob, out_hbm.at[pl.ds(0, CH)], sem).wait()

    def compute_chunk(c, rs, rd, ob):
        def group_body(g, _):
            for ee in range(16):
                e = g * 16 + ee
                s = rs[e, pl.ds(0, 32)]
                d = rd[e, pl.ds(0, 32)]
                sa, sb = plsc.unpack(s, format=plsc.PackFormat.INTERLEAVED)
                da, db = plsc.unpack(d, format=plsc.PackFormat.INTERLEAVED)
                acc0 = sa * da
                acc1 = sb * db
                for k in range(1, D // 32):
                    s = rs[e, pl.ds(k * 32, 32)]
                    d = rd[e, pl.ds(k * 32, 32)]
                    sa, sb = plsc.unpack(s, format=plsc.PackFormat.INTERLEAVED)
                    da, db = plsc.unpack(d, format=plsc.PackFormat.INTERLEAVED)
                    acc0 = acc0 + sa * da
                    acc1 = acc1 + sb * db
                m_v[ee, pl.ds(0, 16)] = acc0 + acc1
            tot = plsc.load_gather(m_v, [lane, cols[0]])
            for k in range(1, 16):
                tot = tot + plsc.load_gather(m_v, [lane, cols[k]])
            ob[pl.ds(g * 16, 16)] = tot
            return _

        lax.fori_loop(0, CH // 16, group_body, None)

    NH = NCHUNK // 2
    issue(0, rows_s0, rows_d0, gsem0)
    issue(1, rows_s1, rows_d1, gsem1)

    def pair_body(j2, _):
        c0 = 2 * j2
        c1 = c0 + 1
        wait_rows(rows_s0, rows_d0, gsem0)

        @pl.when(j2 > 0)
        def _w0():
            wait_out(out0, osem0)

        compute_chunk(c0, rows_s0, rows_d0, out0)
        pltpu.async_copy(out0, out_hbm.at[pl.ds(base + _chunk_off(c0), CH)],
                         osem0)

        @pl.when(j2 < NH - 1)
        def _i0():
            issue(c0 + 2, rows_s0, rows_d0, gsem0)

        wait_rows(rows_s1, rows_d1, gsem1)

        @pl.when(j2 > 0)
        def _w1():
            wait_out(out1, osem1)

        compute_chunk(c1, rows_s1, rows_d1, out1)
        pltpu.async_copy(out1, out_hbm.at[pl.ds(base + _chunk_off(c1), CH)],
                         osem1)

        @pl.when(j2 < NH - 1)
        def _i1():
            issue(c1 + 2, rows_s1, rows_d1, gsem1)

        return _

    lax.fori_loop(0, NH, pair_body, None)
    wait_out(out0, osem0)
    wait_out(out1, osem1)


@functools.partial(jax.jit, static_argnames=())
def kernel(edge_index, h):
    src = edge_index[0].astype(jnp.int32)
    dst = edge_index[1].astype(jnp.int32)
    hb = h.astype(jnp.bfloat16)

    mesh = plsc.VectorSubcoreMesh(core_axis_name="c", subcore_axis_name="s")
    return pl.kernel(
        _dot_body,
        out_type=jax.ShapeDtypeStruct((N_EDGES,), jnp.float32),
        mesh=mesh,
        compiler_params=pltpu.CompilerParams(use_tc_tiling_on_sc=False,
                                             needs_layout_passes=False),
        scratch_types=[
            pltpu.VMEM((EPW,), jnp.int32),
            pltpu.VMEM((EPW,), jnp.int32),
            pltpu.VMEM((CH, D), jnp.bfloat16),
            pltpu.VMEM((CH, D), jnp.bfloat16),
            pltpu.VMEM((CH, D), jnp.bfloat16),
            pltpu.VMEM((CH, D), jnp.bfloat16),
            pltpu.VMEM((CH,), jnp.float32),
            pltpu.VMEM((CH,), jnp.float32),
            pltpu.VMEM((16, 17), jnp.float32),
            pltpu.SemaphoreType.DMA,
            pltpu.SemaphoreType.DMA,
            pltpu.SemaphoreType.DMA,
            pltpu.SemaphoreType.DMA,
        ],
    )(src, dst, hb)


# speedup vs baseline: 1.0035x; 1.0035x over previous
import functools

import jax
import jax.numpy as jnp
from jax import lax
from jax.experimental import pallas as pl
from jax.experimental.pallas import tpu as pltpu
from jax.experimental.pallas import tpu_sc as plsc

N_NODES = 10000
N_EDGES = 160000
D = 256

NC = 2
NS = 16
NW = NC * NS
EPW = N_EDGES // NW
CH = 128
NCHUNK = 40
TAIL_OFF = EPW - CH


def _chunk_off(c):
    return jnp.minimum(c * CH, TAIL_OFF)


def _dot_body(src_hbm, dst_hbm, h_hbm, out_hbm,
              idx_src_v, idx_dst_v,
              rows_s0, rows_d0, rows_s1, rows_d1,
              out0, out1, m_v,
              gsem0, gsem1, osem0, osem1):
    wid = lax.axis_index("s") * NC + lax.axis_index("c")
    base = wid * EPW
    pltpu.sync_copy(src_hbm.at[pl.ds(base, EPW)], idx_src_v)
    pltpu.sync_copy(dst_hbm.at[pl.ds(base, EPW)], idx_dst_v)

    lane = lax.iota(jnp.int32, 16)
    cols = [jnp.full((16,), k, jnp.int32) for k in range(16)]

    def issue(c, rs, rd, sem):
        off = _chunk_off(c)
        pltpu.async_copy(h_hbm.at[idx_src_v.at[pl.ds(off, CH)]], rs, sem)
        pltpu.async_copy(h_hbm.at[idx_dst_v.at[pl.ds(off, CH)]], rd, sem)

    def wait_rows(rs, rd, sem):
        dummy = h_hbm.at[pl.ds(0, CH), :]
        pltpu.make_async_copy(dummy, rs, sem).wait()
        pltpu.make_async_copy(dummy, rd, sem).wait()

    def wait_out(ob, sem):
        pltpu.make_async_copy(ob, out_hbm.at[pl.ds(0, CH)], sem).wait()

    def compute_chunk(c, rs, rd, ob):
        def group_body(g, _):
            for ee in range(16):
                e = g * 16 + ee
                s = rs[e, pl.ds(0, 32)]
                d = rd[e, pl.ds(0, 32)]
                sa, sb = plsc.unpack(s, format=plsc.PackFormat.INTERLEAVED)
                da, db = plsc.unpack(d, format=plsc.PackFormat.INTERLEAVED)
                acc0 = sa * da
                acc1 = sb * db
                for k in range(1, D // 32):
                    s = rs[e, pl.ds(k * 32, 32)]
                    d = rd[e, pl.ds(k * 32, 32)]
                    sa, sb = plsc.unpack(s, format=plsc.PackFormat.INTERLEAVED)
                    da, db = plsc.unpack(d, format=plsc.PackFormat.INTERLEAVED)
                    acc0 = acc0 + sa * da
                    acc1 = acc1 + sb * db
                m_v[ee, pl.ds(0, 16)] = acc0 + acc1
            tot = plsc.load_gather(m_v, [lane, cols[0]])
            for k in range(1, 16):
                tot = tot + plsc.load_gather(m_v, [lane, cols[k]])
            ob[pl.ds(g * 16, 16)] = tot
            return _

        lax.fori_loop(0, CH // 16, group_body, None)

    NH = NCHUNK // 2
    issue(0, rows_s0, rows_d0, gsem0)
    issue(1, rows_s1, rows_d1, gsem1)

    def pair_body(j2, _):
        c0 = 2 * j2
        c1 = c0 + 1
        wait_rows(rows_s0, rows_d0, gsem0)

        @pl.when(j2 > 0)
        def _w0():
            wait_out(out0, osem0)

        compute_chunk(c0, rows_s0, rows_d0, out0)
        pltpu.async_copy(out0, out_hbm.at[pl.ds(base + _chunk_off(c0), CH)],
                         osem0)

        @pl.when(j2 < NH - 1)
        def _i0():
            issue(c0 + 2, rows_s0, rows_d0, gsem0)

        wait_rows(rows_s1, rows_d1, gsem1)

        @pl.when(j2 > 0)
        def _w1():
            wait_out(out1, osem1)

        compute_chunk(c1, rows_s1, rows_d1, out1)
        pltpu.async_copy(out1, out_hbm.at[pl.ds(base + _chunk_off(c1), CH)],
                         osem1)

        @pl.when(j2 < NH - 1)
        def _i1():
            issue(c1 + 2, rows_s1, rows_d1, gsem1)

        return _

    lax.fori_loop(0, NH, pair_body, None)
    wait_out(out0, osem0)
    wait_out(out1, osem1)


@functools.partial(jax.jit, static_argnames=())
def kernel(edge_index, h):
    src = edge_index[0].astype(jnp.int32)
    dst = edge_index[1].astype(jnp.int32)
    hb = h.astype(jnp.bfloat16)

    mesh = plsc.VectorSubcoreMesh(core_axis_name="c", subcore_axis_name="s")
    return pl.kernel(
        _dot_body,
        out_type=jax.ShapeDtypeStruct((N_EDGES,), jnp.float32),
        mesh=mesh,
        compiler_params=pltpu.CompilerParams(use_tc_tiling_on_sc=False,
                                             needs_layout_passes=False),
        scratch_types=[
            pltpu.VMEM((EPW,), jnp.int32),
            pltpu.VMEM((EPW,), jnp.int32),
            pltpu.VMEM((CH, D), jnp.bfloat16),
            pltpu.VMEM((CH, D), jnp.bfloat16),
            pltpu.VMEM((CH, D), jnp.bfloat16),
            pltpu.VMEM((CH, D), jnp.bfloat16),
            pltpu.VMEM((CH,), jnp.float32),
            pltpu.VMEM((CH,), jnp.float32),
            pltpu.VMEM((16, 17), jnp.float32),
            pltpu.SemaphoreType.DMA,
            pltpu.SemaphoreType.DMA,
            pltpu.SemaphoreType.DMA,
            pltpu.SemaphoreType.DMA,
        ],
    )(src, dst, hb)
